# Initial kernel scaffold; baseline (speedup 1.0000x reference)
import jax
import jax.numpy as jnp
from jax.experimental import pallas as pl


def kernel(boxes, classification, translation, rotation):
    raise NotImplementedError("write your pallas kernel here")


# speedup vs baseline: 34.5298x; 34.5298x over previous
import jax
import jax.numpy as jnp
from jax import lax
from jax.experimental import pallas as pl
from jax.experimental.pallas import tpu as pltpu

NEG_V = -1e30
SCORE_T = 0.01
NMS_T = 0.5
MAX_DET = 100
N_BOX = 20000
N_CLS = 8
LANES = 128


def _fd_kernel(scoresT_ref, boxesT_ref, data_ref,
               out_scores_ref, out_labels_ref, out_mat_ref, s_ref):
    x1 = boxesT_ref[0:1, :]
    y1 = boxesT_ref[1:2, :]
    x2 = boxesT_ref[2:3, :]
    y2 = boxesT_ref[3:4, :]
    areas = (x2 - x1) * (y2 - y1)

    sc = scoresT_ref[:]
    s_ref[:] = jnp.where(sc > SCORE_T, sc, NEG_V)

    iota_n = lax.broadcasted_iota(jnp.int32, (N_CLS, N_BOX), 1)
    lane_iota = lax.broadcasted_iota(jnp.int32, (N_CLS, LANES), 1)

    def nms_body(k, carry):
        vals, idxs = carry
        s = s_ref[:]
        val = jnp.max(s, axis=1, keepdims=True)
        eq = s == val
        idx = jnp.min(jnp.where(eq, iota_n, jnp.int32(2 ** 30)),
                      axis=1, keepdims=True)
        onehot = iota_n == idx
        bx1 = jnp.max(jnp.where(onehot, x1, NEG_V), axis=1, keepdims=True)
        by1 = jnp.max(jnp.where(onehot, y1, NEG_V), axis=1, keepdims=True)
        bx2 = jnp.max(jnp.where(onehot, x2, NEG_V), axis=1, keepdims=True)
        by2 = jnp.max(jnp.where(onehot, y2, NEG_V), axis=1, keepdims=True)
        ba = (bx2 - bx1) * (by2 - by1)
        xx1 = jnp.maximum(x1, bx1)
        yy1 = jnp.maximum(y1, by1)
        xx2 = jnp.minimum(x2, bx2)
        yy2 = jnp.minimum(y2, by2)
        inter = jnp.maximum(xx2 - xx1, 0.0) * jnp.maximum(yy2 - yy1, 0.0)
        iou = inter / (areas + ba - inter + 1e-9)
        s = jnp.where(iou > NMS_T, NEG_V, s)
        s = jnp.where(onehot, NEG_V, s)
        s_ref[:] = s
        here = lane_iota == k
        vals = jnp.where(here, val, vals)
        idxs = jnp.where(here, idx, idxs)
        return vals, idxs

    vals0 = jnp.full((N_CLS, LANES), NEG_V, jnp.float32)
    idxs0 = jnp.zeros((N_CLS, LANES), jnp.int32)
    vals, idxs = lax.fori_loop(0, MAX_DET, nms_body, (vals0, idxs0))

    cand = jnp.where(vals > NEG_V / 2, vals, NEG_V)
    c_iota = lax.broadcasted_iota(jnp.int32, (N_CLS, LANES), 0)
    key = c_iota * MAX_DET + lane_iota
    row_iota = lax.broadcasted_iota(jnp.int32, (LANES, 1), 0)

    def topk_body(t, carry):
        cand, tval, tlab, tidx = carry
        m = jnp.max(cand, axis=(0, 1), keepdims=True)
        eqm = cand == m
        selkey = jnp.min(jnp.where(eqm, key, jnp.int32(2 ** 30)),
                         axis=(0, 1), keepdims=True)
        oh = eqm & (key == selkey)
        lab = jnp.max(jnp.where(oh, c_iota, -1), axis=(0, 1), keepdims=True)
        oidx = jnp.max(jnp.where(oh, idxs, -1), axis=(0, 1), keepdims=True)
        cand = jnp.where(oh, NEG_V, cand)
        here = row_iota == t
        tval = jnp.where(here, m, tval)
        tlab = jnp.where(here, lab, tlab)
        tidx = jnp.where(here, oidx, tidx)
        return cand, tval, tlab, tidx

    tval0 = jnp.full((LANES, 1), NEG_V, jnp.float32)
    tlab0 = jnp.zeros((LANES, 1), jnp.int32)
    tidx0 = jnp.zeros((LANES, 1), jnp.int32)
    _, tval, tlab, tidx = lax.fori_loop(
        0, MAX_DET, topk_body, (cand, tval0, tlab0, tidx0))

    valid = tval > NEG_V / 2
    out_scores_ref[:] = jnp.where(valid, tval, -1.0)
    out_labels_ref[:] = jnp.where(valid, tlab, -1)
    col_iota = lax.broadcasted_iota(jnp.int32, (LANES, N_BOX), 1)
    mat = ((col_iota == tidx) & valid).astype(jnp.float32)
    gathered = lax.dot_general(mat, data_ref[:],
                               (((1,), (0,)), ((), ())),
                               precision=lax.Precision.HIGHEST,
                               preferred_element_type=jnp.float32)
    out_mat_ref[:] = jnp.where(valid, gathered, -1.0)


def _filter_detections_single(boxes, classification, translation, rotation):
    scoresT = classification.T
    boxesT = boxes.T
    data = jnp.concatenate(
        [boxes, rotation, translation,
         jnp.zeros((N_BOX, 6), jnp.float32)], axis=1)

    out_scores, out_labels, out_mat = pl.pallas_call(
        _fd_kernel,
        out_shape=(
            jax.ShapeDtypeStruct((LANES, 1), jnp.float32),
            jax.ShapeDtypeStruct((LANES, 1), jnp.int32),
            jax.ShapeDtypeStruct((LANES, 16), jnp.float32),
        ),
        scratch_shapes=[pltpu.VMEM((N_CLS, N_BOX), jnp.float32)],
    )(scoresT, boxesT, data)

    b = out_mat[:MAX_DET, 0:4]
    r = out_mat[:MAX_DET, 4:7]
    t = out_mat[:MAX_DET, 7:10]
    s = out_scores[:MAX_DET, 0]
    l = out_labels[:MAX_DET, 0]
    return b, s, l, r, t


def kernel(boxes, classification, translation, rotation):
    B = boxes.shape[0]
    obs, oss, ols, ors, ots = [], [], [], [], []
    for i in range(B):
        b, s, l, r, t = _filter_detections_single(
            boxes[i], classification[i], translation[i], rotation[i])
        obs.append(b); oss.append(s); ols.append(l); ors.append(r); ots.append(t)
    return (jnp.stack(obs), jnp.stack(oss), jnp.stack(ols),
            jnp.stack(ors), jnp.stack(ots))


# speedup vs baseline: 40.4398x; 1.1712x over previous
import jax
import jax.numpy as jnp
from jax import lax
from jax.experimental import pallas as pl
from jax.experimental.pallas import tpu as pltpu

NEG_V = -1e30
SCORE_T = 0.01
NMS_T = 0.5
MAX_DET = 100
N_BOX = 20000
N_CLS = 8
LANES = 128


def _fd_kernel(scoresT_ref, boxesT_ref, dataT_ref,
               out_scores_ref, out_labels_ref, out_mat_ref, s_ref):
    x1 = boxesT_ref[0:1, :]
    y1 = boxesT_ref[1:2, :]
    x2 = boxesT_ref[2:3, :]
    y2 = boxesT_ref[3:4, :]
    areas = (x2 - x1) * (y2 - y1)

    sc = scoresT_ref[:]
    s_ref[:] = jnp.where(sc > SCORE_T, sc, NEG_V)

    iota_n = lax.broadcasted_iota(jnp.int32, (N_CLS, N_BOX), 1)
    lane_iota = lax.broadcasted_iota(jnp.int32, (N_CLS, LANES), 1)

    def nms_body(k, carry):
        vals, idxs = carry
        s = s_ref[:]
        idx = jnp.argmax(s, axis=1).reshape(N_CLS, 1)
        onehot = iota_n == idx
        val = jnp.max(jnp.where(onehot, s, NEG_V), axis=1, keepdims=True)
        bx1 = jnp.max(jnp.where(onehot, x1, NEG_V), axis=1, keepdims=True)
        by1 = jnp.max(jnp.where(onehot, y1, NEG_V), axis=1, keepdims=True)
        bx2 = jnp.max(jnp.where(onehot, x2, NEG_V), axis=1, keepdims=True)
        by2 = jnp.max(jnp.where(onehot, y2, NEG_V), axis=1, keepdims=True)
        ba = (bx2 - bx1) * (by2 - by1)
        xx1 = jnp.maximum(x1, bx1)
        yy1 = jnp.maximum(y1, by1)
        xx2 = jnp.minimum(x2, bx2)
        yy2 = jnp.minimum(y2, by2)
        inter = jnp.maximum(xx2 - xx1, 0.0) * jnp.maximum(yy2 - yy1, 0.0)
        iou = inter / (areas + ba - inter + 1e-9)
        s = jnp.where((iou > NMS_T) | onehot, NEG_V, s)
        s_ref[:] = s
        here = lane_iota == k
        vals = jnp.where(here, val, vals)
        idxs = jnp.where(here, idx, idxs)
        return vals, idxs

    vals0 = jnp.full((N_CLS, LANES), NEG_V, jnp.float32)
    idxs0 = jnp.zeros((N_CLS, LANES), jnp.int32)
    vals, idxs = lax.fori_loop(0, MAX_DET, nms_body, (vals0, idxs0))

    cand = jnp.where(vals > NEG_V / 2, vals, NEG_V)
    c8 = lax.broadcasted_iota(jnp.int32, (N_CLS, 1), 0)
    lane1 = lax.broadcasted_iota(jnp.int32, (1, LANES), 1)

    def merge_body(t, carry):
        ptr, head, head_idx, tval, tlab, tidx = carry
        m = jnp.max(head, axis=(0, 1), keepdims=True)
        cw = jnp.min(jnp.where(head == m, c8, N_CLS), axis=(0, 1),
                     keepdims=True)
        isw = c8 == cw
        oidx = jnp.max(jnp.where(isw, head_idx, -1), axis=(0, 1),
                       keepdims=True)
        here = lane1 == t
        tval = jnp.where(here, m, tval)
        tlab = jnp.where(here, cw, tlab)
        tidx = jnp.where(here, oidx, tidx)
        ptr = ptr + isw.astype(jnp.int32)
        sel = lane_iota == ptr
        nh = jnp.max(jnp.where(sel, cand, NEG_V), axis=1, keepdims=True)
        nhi = jnp.max(jnp.where(sel, idxs, -1), axis=1, keepdims=True)
        head = jnp.where(isw, nh, head)
        head_idx = jnp.where(isw, nhi, head_idx)
        return ptr, head, head_idx, tval, tlab, tidx

    ptr0 = jnp.zeros((N_CLS, 1), jnp.int32)
    head0 = cand[:, 0:1]
    head_idx0 = idxs[:, 0:1]
    tval0 = jnp.full((1, LANES), NEG_V, jnp.float32)
    tlab0 = jnp.zeros((1, LANES), jnp.int32)
    tidx0 = jnp.zeros((1, LANES), jnp.int32)
    _, _, _, tval, tlab, tidx = lax.fori_loop(
        0, MAX_DET, merge_body,
        (ptr0, head0, head_idx0, tval0, tlab0, tidx0))

    valid = tval > NEG_V / 2
    out_scores_ref[:] = jnp.where(valid, tval, -1.0)
    out_labels_ref[:] = jnp.where(valid, tlab, -1)
    row_iota = lax.broadcasted_iota(jnp.int32, (N_BOX, LANES), 0)
    mat = ((row_iota == tidx) & valid).astype(jnp.float32)
    gathered = lax.dot_general(dataT_ref[:], mat,
                               (((1,), (0,)), ((), ())),
                               precision=lax.Precision.HIGHEST,
                               preferred_element_type=jnp.float32)
    out_mat_ref[:] = jnp.where(valid, gathered, -1.0)


def _filter_detections_single(boxes, classification, translation, rotation):
    scoresT = classification.T
    boxesT = boxes.T
    dataT = jnp.concatenate(
        [boxesT, rotation.T, translation.T,
         jnp.zeros((6, N_BOX), jnp.float32)], axis=0)

    out_scores, out_labels, out_mat = pl.pallas_call(
        _fd_kernel,
        out_shape=(
            jax.ShapeDtypeStruct((1, LANES), jnp.float32),
            jax.ShapeDtypeStruct((1, LANES), jnp.int32),
            jax.ShapeDtypeStruct((16, LANES), jnp.float32),
        ),
        scratch_shapes=[pltpu.VMEM((N_CLS, N_BOX), jnp.float32)],
    )(scoresT, boxesT, dataT)

    g = out_mat[:, :MAX_DET].T
    b = g[:, 0:4]
    r = g[:, 4:7]
    t = g[:, 7:10]
    s = out_scores[0, :MAX_DET]
    l = out_labels[0, :MAX_DET]
    return b, s, l, r, t


def kernel(boxes, classification, translation, rotation):
    B = boxes.shape[0]
    obs, oss, ols, ors, ots = [], [], [], [], []
    for i in range(B):
        b, s, l, r, t = _filter_detections_single(
            boxes[i], classification[i], translation[i], rotation[i])
        obs.append(b); oss.append(s); ols.append(l); ors.append(r); ots.append(t)
    return (jnp.stack(obs), jnp.stack(oss), jnp.stack(ols),
            jnp.stack(ors), jnp.stack(ots))


# speedup vs baseline: 44.5064x; 1.1006x over previous
import jax
import jax.numpy as jnp
from jax import lax
from jax.experimental import pallas as pl
from jax.experimental.pallas import tpu as pltpu

NEG_V = -1e30
SCORE_T = 0.01
NMS_T = 0.5
MAX_DET = 100
N_BOX = 20000
N_CLS = 8
LANES = 128


def _fd_kernel(scoresT_ref, boxesT_ref, dataT_ref,
               out_scores_ref, out_labels_ref, out_mat_ref,
               s_ref, x1_ref, y1_ref, x2_ref, y2_ref, ar_ref, io_ref):
    ones = jnp.ones((N_CLS, 1), jnp.float32)
    x1_ref[:] = ones * boxesT_ref[0:1, :]
    y1_ref[:] = ones * boxesT_ref[1:2, :]
    x2_ref[:] = ones * boxesT_ref[2:3, :]
    y2_ref[:] = ones * boxesT_ref[3:4, :]
    ar_ref[:] = (x2_ref[:] - x1_ref[:]) * (y2_ref[:] - y1_ref[:])
    io_ref[:] = lax.broadcasted_iota(jnp.int32, (N_CLS, N_BOX), 1)

    sc = scoresT_ref[:]
    s0 = jnp.where(sc > SCORE_T, sc, NEG_V)
    s_ref[:] = s0
    idx0 = jnp.argmax(s0, axis=1).reshape(N_CLS, 1)
    val0 = jnp.max(s0, axis=1, keepdims=True)

    lane_iota = lax.broadcasted_iota(jnp.int32, (N_CLS, LANES), 1)

    def nms_body(k, carry):
        val, idx, vals, idxs = carry
        iota_n = io_ref[:]
        onehot = iota_n == idx
        x1 = x1_ref[:]
        y1 = y1_ref[:]
        x2 = x2_ref[:]
        y2 = y2_ref[:]
        bx1 = jnp.max(jnp.where(onehot, x1, NEG_V), axis=1, keepdims=True)
        by1 = jnp.max(jnp.where(onehot, y1, NEG_V), axis=1, keepdims=True)
        bx2 = jnp.max(jnp.where(onehot, x2, NEG_V), axis=1, keepdims=True)
        by2 = jnp.max(jnp.where(onehot, y2, NEG_V), axis=1, keepdims=True)
        ba = (bx2 - bx1) * (by2 - by1)
        xx1 = jnp.maximum(x1, bx1)
        yy1 = jnp.maximum(y1, by1)
        xx2 = jnp.minimum(x2, bx2)
        yy2 = jnp.minimum(y2, by2)
        inter = jnp.maximum(xx2 - xx1, 0.0) * jnp.maximum(yy2 - yy1, 0.0)
        iou = inter / (ar_ref[:] + ba - inter + 1e-9)
        s_new = jnp.where((iou > NMS_T) | onehot, NEG_V, s_ref[:])
        s_ref[:] = s_new
        idx_n = jnp.argmax(s_new, axis=1).reshape(N_CLS, 1)
        val_n = jnp.max(s_new, axis=1, keepdims=True)
        here = lane_iota == k
        vals = jnp.where(here, val, vals)
        idxs = jnp.where(here, idx, idxs)
        return val_n, idx_n, vals, idxs

    vals0 = jnp.full((N_CLS, LANES), NEG_V, jnp.float32)
    idxs0 = jnp.zeros((N_CLS, LANES), jnp.int32)
    _, _, vals, idxs = lax.fori_loop(
        0, MAX_DET, nms_body, (val0, idx0, vals0, idxs0))

    cand = jnp.where(vals > NEG_V / 2, vals, NEG_V)
    c8 = lax.broadcasted_iota(jnp.int32, (N_CLS, 1), 0)
    lane1 = lax.broadcasted_iota(jnp.int32, (1, LANES), 1)

    def merge_body(t, carry):
        ptr, head, head_idx, tval, tlab, tidx = carry
        m = jnp.max(head, axis=(0, 1), keepdims=True)
        cw = jnp.min(jnp.where(head == m, c8, N_CLS), axis=(0, 1),
                     keepdims=True)
        isw = c8 == cw
        oidx = jnp.max(jnp.where(isw, head_idx, -1), axis=(0, 1),
                       keepdims=True)
        here = lane1 == t
        tval = jnp.where(here, m, tval)
        tlab = jnp.where(here, cw, tlab)
        tidx = jnp.where(here, oidx, tidx)
        ptr = ptr + isw.astype(jnp.int32)
        sel = lane_iota == ptr
        nh = jnp.max(jnp.where(sel, cand, NEG_V), axis=1, keepdims=True)
        nhi = jnp.max(jnp.where(sel, idxs, -1), axis=1, keepdims=True)
        head = jnp.where(isw, nh, head)
        head_idx = jnp.where(isw, nhi, head_idx)
        return ptr, head, head_idx, tval, tlab, tidx

    ptr0 = jnp.zeros((N_CLS, 1), jnp.int32)
    head0 = cand[:, 0:1]
    head_idx0 = idxs[:, 0:1]
    tval0 = jnp.full((1, LANES), NEG_V, jnp.float32)
    tlab0 = jnp.zeros((1, LANES), jnp.int32)
    tidx0 = jnp.zeros((1, LANES), jnp.int32)
    _, _, _, tval, tlab, tidx = lax.fori_loop(
        0, MAX_DET, merge_body,
        (ptr0, head0, head_idx0, tval0, tlab0, tidx0))

    valid = tval > NEG_V / 2
    out_scores_ref[:] = jnp.where(valid, tval, -1.0)
    out_labels_ref[:] = jnp.where(valid, tlab, -1)
    row_iota = lax.broadcasted_iota(jnp.int32, (N_BOX, LANES), 0)
    mat = ((row_iota == tidx) & valid).astype(jnp.float32)
    gathered = lax.dot_general(dataT_ref[:], mat,
                               (((1,), (0,)), ((), ())),
                               precision=lax.Precision.HIGHEST,
                               preferred_element_type=jnp.float32)
    out_mat_ref[:] = jnp.where(valid, gathered, -1.0)


def _filter_detections_single(boxes, classification, translation, rotation):
    scoresT = classification.T
    boxesT = boxes.T
    dataT = jnp.concatenate(
        [boxesT, rotation.T, translation.T,
         jnp.zeros((6, N_BOX), jnp.float32)], axis=0)

    out_scores, out_labels, out_mat = pl.pallas_call(
        _fd_kernel,
        out_shape=(
            jax.ShapeDtypeStruct((1, LANES), jnp.float32),
            jax.ShapeDtypeStruct((1, LANES), jnp.int32),
            jax.ShapeDtypeStruct((16, LANES), jnp.float32),
        ),
        scratch_shapes=[pltpu.VMEM((N_CLS, N_BOX), jnp.float32),
                        pltpu.VMEM((N_CLS, N_BOX), jnp.float32),
                        pltpu.VMEM((N_CLS, N_BOX), jnp.float32),
                        pltpu.VMEM((N_CLS, N_BOX), jnp.float32),
                        pltpu.VMEM((N_CLS, N_BOX), jnp.float32),
                        pltpu.VMEM((N_CLS, N_BOX), jnp.float32),
                        pltpu.VMEM((N_CLS, N_BOX), jnp.int32)],
    )(scoresT, boxesT, dataT)

    g = out_mat[:, :MAX_DET].T
    b = g[:, 0:4]
    r = g[:, 4:7]
    t = g[:, 7:10]
    s = out_scores[0, :MAX_DET]
    l = out_labels[0, :MAX_DET]
    return b, s, l, r, t


def kernel(boxes, classification, translation, rotation):
    B = boxes.shape[0]
    obs, oss, ols, ors, ots = [], [], [], [], []
    for i in range(B):
        b, s, l, r, t = _filter_detections_single(
            boxes[i], classification[i], translation[i], rotation[i])
        obs.append(b); oss.append(s); ols.append(l); ors.append(r); ots.append(t)
    return (jnp.stack(obs), jnp.stack(oss), jnp.stack(ols),
            jnp.stack(ors), jnp.stack(ots))
